# Initial kernel scaffold; baseline (speedup 1.0000x reference)
import jax
import jax.numpy as jnp
from jax.experimental import pallas as pl


def kernel(bin_ids, table):
    raise NotImplementedError("write your pallas kernel here")


# speedup vs baseline: 1.4351x; 1.4351x over previous
import functools

import jax
import jax.numpy as jnp
from jax import lax
from jax.experimental import pallas as pl
from jax.experimental.pallas import tpu as pltpu
from jax.experimental.pallas import tpu_sc as plsc

BATCH = 16384
FIELDS = 26
EMBED_DIM = 32
B = BATCH * FIELDS
NC, NS = 2, 16
NW = NC * NS
CHUNK = 128
J = B // (NW * CHUNK)

_mesh = plsc.VectorSubcoreMesh(core_axis_name="c", subcore_axis_name="s")


@functools.partial(
    pl.kernel,
    mesh=_mesh,
    out_type=jax.ShapeDtypeStruct((B, EMBED_DIM), jnp.float32),
    scratch_types=[
        pltpu.VMEM((J, CHUNK), jnp.int32),
        pltpu.VMEM((CHUNK, EMBED_DIM), jnp.float32),
        pltpu.SemaphoreType.DMA,
    ],
    compiler_params=pltpu.CompilerParams(use_tc_tiling_on_sc=False),
)
def _gather_kernel(idx_hbm, table_hbm, out_hbm, idx_v, rows_v, sem):
    wid = lax.axis_index("s") * NC + lax.axis_index("c")
    pltpu.sync_copy(idx_hbm.at[pl.ds(wid * J, J)], idx_v)

    def step(j, carry):
        pltpu.async_copy(table_hbm.at[idx_v.at[j]], rows_v, sem).wait()
        pltpu.sync_copy(rows_v, out_hbm.at[pl.ds((wid * J + j) * CHUNK, CHUNK)])
        return carry

    lax.fori_loop(0, J, step, 0)


def kernel(bin_ids, table):
    idx = bin_ids.reshape(NW * J, CHUNK)
    out = _gather_kernel(idx, table)
    return out.reshape(BATCH, FIELDS, EMBED_DIM)


# speedup vs baseline: 1.5741x; 1.0968x over previous
import functools

import jax
import jax.numpy as jnp
from jax import lax
from jax.experimental import pallas as pl
from jax.experimental.pallas import tpu as pltpu
from jax.experimental.pallas import tpu_sc as plsc

BATCH = 16384
FIELDS = 26
EMBED_DIM = 32
B = BATCH * FIELDS
NC, NS = 2, 16
NW = NC * NS
CHUNK = 128
J = B // (NW * CHUNK)
K = 4
S = J // K

_mesh = plsc.VectorSubcoreMesh(core_axis_name="c", subcore_axis_name="s")


@functools.partial(
    pl.kernel,
    mesh=_mesh,
    out_type=jax.ShapeDtypeStruct((B, EMBED_DIM), jnp.float32),
    scratch_types=[
        pltpu.VMEM((J, CHUNK), jnp.int32),
        pltpu.VMEM((2, K * CHUNK, EMBED_DIM), jnp.float32),
        pltpu.SemaphoreType.DMA,
        pltpu.SemaphoreType.DMA,
        pltpu.SemaphoreType.DMA,
        pltpu.SemaphoreType.DMA,
    ],
    compiler_params=pltpu.CompilerParams(use_tc_tiling_on_sc=False),
)
def _gather_kernel(idx_hbm, table_hbm, out_hbm, idx_v, rows_v, g0, g1, w0, w1):
    wid = lax.axis_index("s") * NC + lax.axis_index("c")
    pltpu.sync_copy(idx_hbm.at[pl.ds(wid * J, J)], idx_v)
    gsems = (g0, g1)
    wsems = (w0, w1)

    def fire(s, b):
        for k in range(K):
            pltpu.async_copy(
                table_hbm.at[idx_v.at[s * K + k]],
                rows_v.at[b].at[pl.ds(k * CHUNK, CHUNK)],
                gsems[b],
            )

    def drain_g(b):
        pltpu.make_async_copy(
            out_hbm.at[pl.ds(0, K * CHUNK)], rows_v.at[b], gsems[b]
        ).wait()

    def write(s, b):
        pltpu.async_copy(
            rows_v.at[b],
            out_hbm.at[pl.ds((wid * J + s * K) * CHUNK, K * CHUNK)],
            wsems[b],
        )

    def wait_w(b):
        pltpu.make_async_copy(
            rows_v.at[b], out_hbm.at[pl.ds(0, K * CHUNK)], wsems[b]
        ).wait()

    fire(0, 0)
    fire(1, 1)
    drain_g(0)
    write(0, 0)
    wait_w(0)
    fire(2, 0)
    drain_g(1)
    write(1, 1)

    def pair(i, carry):
        s0 = 2 * i
        wait_w(1)
        fire(s0 + 1, 1)
        drain_g(0)
        write(s0, 0)
        wait_w(0)

        @pl.when(i < S // 2 - 1)
        def _():
            fire(s0 + 2, 0)

        drain_g(1)
        write(s0 + 1, 1)
        return carry

    lax.fori_loop(1, S // 2, pair, 0)
    wait_w(1)


def kernel(bin_ids, table):
    idx = bin_ids.reshape(NW * J, CHUNK)
    out = _gather_kernel(idx, table)
    return out.reshape(BATCH, FIELDS, EMBED_DIM)
